# baseline (speedup 1.0000x reference)
import jax
import jax.numpy as jnp
from jax.experimental import pallas as pl
from jax.experimental.pallas import tpu as pltpu

_NUM_CHUNKS = 4


def _make_stream_copy(chunk_rows):
    def _stream_copy(x_hbm, o_hbm, *rest):
        bufs = rest[:_NUM_CHUNKS]
        in_sems = rest[_NUM_CHUNKS:2 * _NUM_CHUNKS]
        out_sems = rest[2 * _NUM_CHUNKS:]
        ins = [
            pltpu.make_async_copy(
                x_hbm.at[pl.ds(i * chunk_rows, chunk_rows), :],
                bufs[i], in_sems[i])
            for i in range(_NUM_CHUNKS)
        ]
        outs = [
            pltpu.make_async_copy(
                bufs[i], o_hbm.at[pl.ds(i * chunk_rows, chunk_rows), :],
                out_sems[i])
            for i in range(_NUM_CHUNKS)
        ]
        ins[0].start()
        ins[1].start()
        for i in range(_NUM_CHUNKS):
            ins[i].wait()
            outs[i].start()
            if i + 2 < _NUM_CHUNKS:
                ins[i + 2].start()
        for c in outs:
            c.wait()

    return _stream_copy


def kernel(x, idx, label):
    del idx, label
    rows, cols = x.shape
    chunk_rows = rows // _NUM_CHUNKS
    return pl.pallas_call(
        _make_stream_copy(chunk_rows),
        out_shape=jax.ShapeDtypeStruct(x.shape, x.dtype),
        in_specs=[pl.BlockSpec(memory_space=pl.ANY)],
        out_specs=pl.BlockSpec(memory_space=pl.ANY),
        scratch_shapes=(
            [pltpu.VMEM((chunk_rows, cols), x.dtype)] * _NUM_CHUNKS
            + [pltpu.SemaphoreType.DMA] * (2 * _NUM_CHUNKS)
        ),
    )(x)


# speedup vs baseline: 1.3652x; 1.3652x over previous
import jax
import jax.numpy as jnp
from jax.experimental import pallas as pl
from jax.experimental.pallas import tpu as pltpu

_NUM_CHUNKS = 4


def _make_stream_copy(chunk_rows):
    def _stream_copy(x_hbm, o_hbm, *rest):
        bufs = rest[:_NUM_CHUNKS]
        in_sems = rest[_NUM_CHUNKS:2 * _NUM_CHUNKS]
        out_sems = rest[2 * _NUM_CHUNKS:]
        ins = [
            pltpu.make_async_copy(
                x_hbm.at[pl.ds(i * chunk_rows, chunk_rows), :],
                bufs[i], in_sems[i])
            for i in range(_NUM_CHUNKS)
        ]
        outs = [
            pltpu.make_async_copy(
                bufs[i], o_hbm.at[pl.ds(i * chunk_rows, chunk_rows), :],
                out_sems[i])
            for i in range(_NUM_CHUNKS)
        ]
        for c in ins:
            c.start()
        for i in range(_NUM_CHUNKS):
            ins[i].wait()
            outs[i].start()
        for c in outs:
            c.wait()

    return _stream_copy


def kernel(x, idx, label):
    del idx, label
    rows, cols = x.shape
    chunk_rows = rows // _NUM_CHUNKS
    return pl.pallas_call(
        _make_stream_copy(chunk_rows),
        out_shape=jax.ShapeDtypeStruct(x.shape, x.dtype),
        in_specs=[pl.BlockSpec(memory_space=pl.ANY)],
        out_specs=pl.BlockSpec(memory_space=pl.ANY),
        scratch_shapes=(
            [pltpu.VMEM((chunk_rows, cols), x.dtype)] * _NUM_CHUNKS
            + [pltpu.SemaphoreType.DMA] * (2 * _NUM_CHUNKS)
        ),
    )(x)
